# Initial kernel scaffold; baseline (speedup 1.0000x reference)
import jax
import jax.numpy as jnp
from jax.experimental import pallas as pl


def kernel(inputs):
    raise NotImplementedError("write your pallas kernel here")


# speedup vs baseline: 48.9826x; 48.9826x over previous
import functools

import jax
import jax.numpy as jnp
from jax import lax
from jax.experimental import pallas as pl
from jax.experimental.pallas import tpu as pltpu
from jax.experimental.pallas import tpu_sc as plsc

B, T, C = 32, 32768, 64
L = 16
G = C // L
R = 256
NCH = T // R
SUB = 8
NSUB = R // SUB
NEG = float("-inf")


def _tree_max(vs):
    while len(vs) > 1:
        nxt = [jnp.maximum(vs[2 * i], vs[2 * i + 1]) for i in range(len(vs) // 2)]
        if len(vs) % 2:
            nxt.append(vs[-1])
        vs = nxt
    return vs[0]


def _build():
    mesh = plsc.VectorSubcoreMesh(core_axis_name="c", subcore_axis_name="s")
    nc = mesh.num_cores

    @functools.partial(
        pl.kernel,
        out_type=jax.ShapeDtypeStruct((B, C), jnp.float32),
        mesh=mesh,
        compiler_params=pltpu.CompilerParams(needs_layout_passes=False),
        scratch_types=[
            pltpu.VMEM((2, R, C), jnp.float32),
            pltpu.VMEM((5 * G, L), jnp.float32),
            pltpu.VMEM((C,), jnp.float32),
            pltpu.SemaphoreType.DMA,
            pltpu.SemaphoreType.DMA,
        ],
    )
    def k(x_hbm, out_hbm, buf, state, outbuf, sem0, sem1):
        b = lax.axis_index("s") * nc + lax.axis_index("c")
        sems = (sem0, sem1)

        def start(ch, which):
            pltpu.async_copy(x_hbm.at[b, pl.ds(ch * R, R)], buf.at[which],
                             sems[which])

        def wait(which):
            pltpu.make_async_copy(x_hbm.at[b, pl.ds(0, R)], buf.at[which],
                                  sems[which]).wait()

        for i in range(5 * G):
            state[i] = jnp.full((L,), NEG, jnp.float32)

        def process(which):
            def sub_body(s, carry):
                base = s * SUB
                for g in range(G):
                    lane = pl.ds(g * L, L)
                    vals = [buf[which, base + r, lane] for r in range(SUB)]
                    mx = _tree_max(vals)
                    thr = state[5 * g + 4]
                    cnt = plsc.all_reduce_population_count(mx > thr)

                    @pl.when(cnt[0] > 0)
                    def _():
                        m = [state[5 * g + i] for i in range(5)]
                        for r in range(SUB):
                            v = vals[r]
                            for i in range(5):
                                hi = jnp.maximum(m[i], v)
                                v = jnp.minimum(m[i], v)
                                m[i] = hi
                        for i in range(5):
                            state[5 * g + i] = m[i]
                return carry

            lax.fori_loop(0, NSUB, sub_body, 0)

        start(0, 0)

        def pair(p, carry):
            start(2 * p + 1, 1)
            wait(0)
            process(0)

            @pl.when(2 * p + 2 < NCH)
            def _():
                start(2 * p + 2, 0)

            wait(1)
            process(1)
            return carry

        lax.fori_loop(0, NCH // 2, pair, 0)

        for g in range(G):
            acc = state[5 * g]
            for i in range(1, 5):
                acc = acc + state[5 * g + i]
            outbuf[pl.ds(g * L, L)] = acc * jnp.float32(0.2)

        pltpu.sync_copy(outbuf, out_hbm.at[b])

    return k


def kernel(inputs):
    return _build()(inputs)
